# Initial kernel scaffold; baseline (speedup 1.0000x reference)
import jax
import jax.numpy as jnp
from jax.experimental import pallas as pl


def kernel(body, mask, rule_idx):
    raise NotImplementedError("write your pallas kernel here")


# speedup vs baseline: 1.4810x; 1.4810x over previous
import functools

import jax
import jax.numpy as jnp
from jax import lax
from jax.experimental import pallas as pl
from jax.experimental.pallas import tpu as pltpu
from jax.experimental.pallas import tpu_sc as plsc

K = 2048
L = 16
NW = 32


def _tec_body(N, rows_per_worker,
              mask_hbm, rule_hbm, body_hbm,
              body_out_hbm, mask_out_hbm, rule_out_hbm,
              mask_v, rule_v, comb_v, bidx_v, body_v, mko_v, rlo_v, sem):
    nchunks = N // L
    wid = lax.axis_index("s") * 2 + lax.axis_index("c")
    iota = lax.iota(jnp.int32, L)

    def do_row(r):
        row = wid * rows_per_worker + r
        pltpu.sync_copy(mask_hbm.at[row], mask_v)
        pltpu.sync_copy(rule_hbm.at[row], rule_v)

        def step(i, carry):
            n1, n0 = carry
            m = mask_v[pl.ds(i * L, L)]
            incl = plsc.cumsum(m)
            excl = incl - m
            s = jnp.sum(m)
            g = i * L + iota
            pos1 = n1 + excl
            pos0 = n0 + (iota - excl)
            m1 = (m > 0) & (pos1 < K)
            m0 = (m == 0) & (pos0 < K)
            plsc.store_scatter(comb_v, [pos1], g, mask=m1)
            plsc.store_scatter(comb_v, [pos0 + K], g, mask=m0)
            return n1 + s, n0 + (L - s)

        n1f, _ = lax.fori_loop(0, nchunks, step, (0, 0))
        n1e = jnp.minimum(n1f, K)

        row_base = row * N
        for t in range(K // L):
            j = t * L + iota
            take1 = j < n1e
            src = jnp.where(take1, j, j - n1e + K)
            sel = plsc.load_gather(comb_v, [src])
            rlo_v[pl.ds(t * L, L)] = plsc.load_gather(rule_v, [sel])
            mko_v[pl.ds(t * L, L)] = take1.astype(jnp.int32)
            bidx_v[t // 8, pl.ds((t % 8) * L, L)] = sel + row_base

        copies = [
            pltpu.make_async_copy(
                body_hbm.at[bidx_v.at[c]],
                body_v.at[pl.ds(c * 128, 128)],
                sem,
            )
            for c in range(K // 128)
        ]
        for cp in copies:
            cp.start()
        for cp in copies:
            cp.wait()

        pltpu.sync_copy(body_v, body_out_hbm.at[row])
        pltpu.sync_copy(mko_v, mask_out_hbm.at[row])
        pltpu.sync_copy(rlo_v, rule_out_hbm.at[row])

    for r in range(rows_per_worker):
        do_row(r)


@jax.jit
def kernel(body, mask, rule_idx):
    B, N, D = body.shape
    rows_per_worker = B // NW
    rdt = rule_idx.dtype
    mask_i = mask.astype(jnp.int32)
    rule_i = rule_idx.astype(jnp.int32)
    body_flat = body.reshape(B * N, D)

    mesh = plsc.VectorSubcoreMesh(
        core_axis_name="c", subcore_axis_name="s", num_cores=2, num_subcores=16
    )
    body_o, mask_o, rule_o = pl.kernel(
        functools.partial(_tec_body, N, rows_per_worker),
        out_type=(
            jax.ShapeDtypeStruct((B, K, D), jnp.float32),
            jax.ShapeDtypeStruct((B, K), jnp.int32),
            jax.ShapeDtypeStruct((B, K), jnp.int32),
        ),
        mesh=mesh,
        compiler_params=pltpu.CompilerParams(
            needs_layout_passes=False, use_tc_tiling_on_sc=False
        ),
        scratch_types=[
            pltpu.VMEM((N,), jnp.int32),
            pltpu.VMEM((N,), jnp.int32),
            pltpu.VMEM((2 * K,), jnp.int32),
            pltpu.VMEM((K // 128, 128), jnp.int32),
            pltpu.VMEM((K, D), jnp.float32),
            pltpu.VMEM((K,), jnp.int32),
            pltpu.VMEM((K,), jnp.int32),
            pltpu.SemaphoreType.DMA,
        ],
    )(mask_i, rule_i, body_flat)
    return body_o, mask_o.astype(jnp.bool_), rule_o.astype(rdt)


# speedup vs baseline: 12.3158x; 8.3161x over previous
import functools

import jax
import jax.numpy as jnp
from jax import lax
from jax.experimental import pallas as pl
from jax.experimental.pallas import tpu as pltpu
from jax.experimental.pallas import tpu_sc as plsc

K = 2048
L = 16
NW = 32
BLK = 8192


def _tec_body(N, D, rows_per_worker,
              mask_hbm, rule_hbm, body_hbm,
              body_out_hbm, mask_out_hbm, rule_out_hbm,
              mask_v, rule_v, plane_v, comb_v, sel_v, mko_v, rlo_v, obt_v):
    nblocks = N // BLK
    chunks_per_block = BLK // L
    wid = lax.axis_index("s") * 2 + lax.axis_index("c")
    iota = lax.iota(jnp.int32, L)

    def do_row(r):
        row = wid * rows_per_worker + r

        def scan_block(b, carry):
            n1, n0, nb = carry

            def live(carry):
                n1, n0 = carry
                pltpu.sync_copy(
                    mask_hbm.at[row, pl.ds(b * BLK, BLK)],
                    mask_v.at[pl.ds(b * BLK, BLK)],
                )

                def step(i, carry):
                    n1, n0 = carry
                    m = mask_v[pl.ds(b * BLK + i * L, L)]
                    incl = plsc.cumsum(m)
                    excl = incl - m
                    s = jnp.sum(m)
                    g = b * BLK + i * L + iota
                    pos1 = n1 + excl
                    pos0 = n0 + (iota - excl)
                    m1 = (m > 0) & (pos1 < K)
                    m0 = (m == 0) & (pos0 < K)
                    plsc.store_scatter(comb_v, [pos1], g, mask=m1)
                    plsc.store_scatter(comb_v, [pos0 + K], g, mask=m0)
                    return n1 + s, n0 + (L - s)

                n1, n0 = lax.fori_loop(0, chunks_per_block, step, (n1, n0))
                return n1, n0, b + 1

            return lax.cond(
                (n1 < K) | (n0 < K), live, lambda c: (c[0], c[1], nb), (n1, n0)
            )

        n1f, _, nb = lax.fori_loop(0, nblocks, scan_block, (0, 0, 0))
        n1e = jnp.minimum(n1f, K)

        def load_rule_block(b, _):
            def live(_):
                pltpu.sync_copy(
                    rule_hbm.at[row, pl.ds(b * BLK, BLK)],
                    rule_v.at[pl.ds(b * BLK, BLK)],
                )
                return 0

            return lax.cond(b < nb, live, lambda x: x, 0)

        lax.fori_loop(0, nblocks, load_rule_block, 0)

        def combine(t, _):
            j = t * L + iota
            take1 = j < n1e
            src = jnp.where(take1, j, j - n1e + K)
            sel = plsc.load_gather(comb_v, [src])
            sel_v[pl.ds(t * L, L)] = sel
            rlo_v[pl.ds(t * L, L)] = plsc.load_gather(rule_v, [sel])
            mko_v[pl.ds(t * L, L)] = take1.astype(jnp.int32)
            return 0

        lax.fori_loop(0, K // L, combine, 0)
        pltpu.sync_copy(mko_v, mask_out_hbm.at[row])
        pltpu.sync_copy(rlo_v, rule_out_hbm.at[row])

        for d in range(D):
            def load_plane_block(b, _):
                def live(_):
                    pltpu.sync_copy(
                        body_hbm.at[row, d, pl.ds(b * BLK, BLK)],
                        plane_v.at[pl.ds(b * BLK, BLK)],
                    )
                    return 0

                return lax.cond(b < nb, live, lambda x: x, 0)

            lax.fori_loop(0, nblocks, load_plane_block, 0)

            def extract(t, _):
                nvec = sel_v[pl.ds(t * L, L)]
                obt_v[pl.ds(t * L, L)] = plsc.load_gather(plane_v, [nvec])
                return 0

            lax.fori_loop(0, K // L, extract, 0)
            pltpu.sync_copy(obt_v, body_out_hbm.at[row, d])

    for r in range(rows_per_worker):
        do_row(r)


@jax.jit
def kernel(body, mask, rule_idx):
    B, N, D = body.shape
    rows_per_worker = B // NW
    rdt = rule_idx.dtype
    mask_i = mask.astype(jnp.int32)
    rule_i = rule_idx.astype(jnp.int32)
    body_t = body.transpose(0, 2, 1)

    mesh = plsc.VectorSubcoreMesh(
        core_axis_name="c", subcore_axis_name="s", num_cores=2, num_subcores=16
    )
    body_o, mask_o, rule_o = pl.kernel(
        functools.partial(_tec_body, N, D, rows_per_worker),
        out_type=(
            jax.ShapeDtypeStruct((B, D, K), jnp.float32),
            jax.ShapeDtypeStruct((B, K), jnp.int32),
            jax.ShapeDtypeStruct((B, K), jnp.int32),
        ),
        mesh=mesh,
        compiler_params=pltpu.CompilerParams(
            needs_layout_passes=False, use_tc_tiling_on_sc=True
        ),
        scratch_types=[
            pltpu.VMEM((N,), jnp.int32),
            pltpu.VMEM((N,), jnp.int32),
            pltpu.VMEM((N,), jnp.float32),
            pltpu.VMEM((2 * K,), jnp.int32),
            pltpu.VMEM((K,), jnp.int32),
            pltpu.VMEM((K,), jnp.int32),
            pltpu.VMEM((K,), jnp.int32),
            pltpu.VMEM((K,), jnp.float32),
        ],
    )(mask_i, rule_i, body_t)
    return body_o.transpose(0, 2, 1), mask_o.astype(jnp.bool_), rule_o.astype(rdt)
